# Initial kernel scaffold; baseline (speedup 1.0000x reference)
import jax
import jax.numpy as jnp
from jax.experimental import pallas as pl


def kernel(rna_norm, rna_counts, rna_libsize, cas_norm, cas_counts, cas_libsize, adt_norm, edge_index, batch_indices, W1, b1, W2, b2, Wg, a_src, a_dst, batch_emb, Wd1, bd1, Wd2, bd2, Wpi, Wdisp, Wmean, Wrec):
    raise NotImplementedError("write your pallas kernel here")


# speedup vs baseline: 15.8737x; 15.8737x over previous
import functools

import jax
import jax.numpy as jnp
from jax import lax
from jax.experimental import pallas as pl
from jax.experimental.pallas import tpu as pltpu
from jax.experimental.pallas import tpu_sc as plsc

N = 50000
E = 800000
D = 128
DL = 50
NB = 4
H1 = 256
H2 = 128
GW = 64
DEN_COL = 50
HALF = N // 2
BLK = 1000
EB = 128
NBLK_TOTAL = E // EB
STRIPE = 1600
CHUNK = 200


def _enc_body(x_ref, w1_ref, b1_ref, w2_ref, b2_ref, wg_ref, asr_ref, adr_ref,
              g_ref, as_ref, ad_ref):
    x = x_ref[...]
    h = jnp.maximum(jnp.dot(x, w1_ref[...], preferred_element_type=jnp.float32)
                    + b1_ref[...], 0.0)
    h = jnp.maximum(jnp.dot(h, w2_ref[...], preferred_element_type=jnp.float32)
                    + b2_ref[...], 0.0)
    g = jnp.dot(h, wg_ref[...], preferred_element_type=jnp.float32)
    col = lax.broadcasted_iota(jnp.int32, (BLK, GW), 1)
    g = jnp.where(col == DEN_COL, 1.0, g)
    g_ref[...] = g
    as_ref[...] = jnp.dot(g, asr_ref[...], preferred_element_type=jnp.float32)
    ad_ref[...] = jnp.dot(g, adr_ref[...], preferred_element_type=jnp.float32)


def _encoder(rna_norm, w1, b1, w2, b2, wg_pad, asrc_pad, adst_pad):
    nb = N // BLK
    full = lambda i: (0, 0)
    return pl.pallas_call(
        _enc_body,
        grid=(nb,),
        in_specs=[
            pl.BlockSpec((BLK, D), lambda i: (i, 0)),
            pl.BlockSpec((D, H1), full),
            pl.BlockSpec((1, H1), full),
            pl.BlockSpec((H1, H2), full),
            pl.BlockSpec((1, H2), full),
            pl.BlockSpec((H2, GW), full),
            pl.BlockSpec((GW, 1), full),
            pl.BlockSpec((GW, 1), full),
        ],
        out_specs=[
            pl.BlockSpec((BLK, GW), lambda i: (i, 0)),
            pl.BlockSpec((BLK, 1), lambda i: (i, 0)),
            pl.BlockSpec((BLK, 1), lambda i: (i, 0)),
        ],
        out_shape=[
            jax.ShapeDtypeStruct((N, GW), jnp.float32),
            jax.ShapeDtypeStruct((N, 1), jnp.float32),
            jax.ShapeDtypeStruct((N, 1), jnp.float32),
        ],
    )(rna_norm, w1, b1, w2, b2, wg_pad, asrc_pad, adst_pad)


def _edge_ex_body(src_h, dst_h, as_h, ad_h, ex_h, asb, adb, srcb, dstb, exb):
    cid = lax.axis_index("c")
    sid = lax.axis_index("s")
    wid = sid * 2 + cid

    pltpu.sync_copy(as_h, asb)
    pltpu.sync_copy(ad_h, adb)

    neg = jnp.full((16,), -3.0e38, jnp.float32)

    def _mx(i, carry):
        ms, md = carry
        ms = jnp.maximum(ms, asb[pl.ds(i * 16, 16)])
        md = jnp.maximum(md, adb[pl.ds(i * 16, 16)])
        return ms, md

    ms_v, md_v = lax.fori_loop(0, N // 16, _mx, (neg, neg))

    def _allmax(vec):
        for k in (8, 4, 2, 1):
            exb[pl.ds(0, 16)] = vec
            idx = jnp.bitwise_xor(lax.iota(jnp.int32, 16), k)
            vec = jnp.maximum(vec, plsc.load_gather(exb, [idx]))
        return vec

    t = _allmax(ms_v) + _allmax(md_v)
    m_shift = jnp.where(t >= 0.0, t, 0.2 * t)

    nfull = NBLK_TOTAL // 32
    nblk = nfull + jnp.where(wid < NBLK_TOTAL - nfull * 32, 1, 0)

    def _blk(b, c):
        base = (b * 32 + wid) * EB
        pltpu.sync_copy(src_h.at[pl.ds(base, EB)], srcb)
        pltpu.sync_copy(dst_h.at[pl.ds(base, EB)], dstb)
        for j in range(EB // 16):
            sv = srcb[pl.ds(j * 16, 16)]
            dv = dstb[pl.ds(j * 16, 16)]
            e = plsc.load_gather(asb, [sv]) + plsc.load_gather(adb, [dv])
            e = jnp.where(e >= 0.0, e, 0.2 * e)
            exb[pl.ds(j * 16, 16)] = jnp.exp(e - m_shift)
        pltpu.sync_copy(exb, ex_h.at[pl.ds(base, EB)])
        return c

    lax.fori_loop(0, nblk, _blk, 0)


def _scatter_body(g_h, src_h, dst_h, ex_h, num_h,
                  srcb, dstb, idxb, exb, rows, zbuf, num_sh, sem):
    cid = lax.axis_index("c")
    sid = lax.axis_index("s")
    lo = cid * HALF

    zeros16 = jnp.zeros((16,), jnp.float32)

    def _zrow(r, c):
        for k in range(GW // 16):
            zbuf[r, pl.ds(k * 16, 16)] = zeros16
        return c

    lax.fori_loop(0, CHUNK, _zrow, 0)

    def _zcp(k, c):
        pltpu.sync_copy(zbuf, num_sh.at[pl.ds(sid * STRIPE + k * CHUNK, CHUNK)])
        return c

    lax.fori_loop(0, STRIPE // CHUNK, _zcp, 0)
    plsc.subcore_barrier()

    nfull = NBLK_TOTAL // 16
    nblk = nfull + jnp.where(sid < NBLK_TOTAL - nfull * 16, 1, 0)

    def _blk(b, c):
        base = (b * 16 + sid) * EB
        pltpu.sync_copy(src_h.at[pl.ds(base, EB)], srcb)
        cp = pltpu.async_copy(g_h.at[srcb], rows, sem)
        pltpu.sync_copy(dst_h.at[pl.ds(base, EB)], dstb)
        pltpu.sync_copy(ex_h.at[pl.ds(base, EB)], exb)
        for j in range(EB // 16):
            dv = dstb[pl.ds(j * 16, 16)]
            msk = (dv >= lo) & (dv < lo + HALF)
            ex = exb[pl.ds(j * 16, 16)]
            exb[pl.ds(j * 16, 16)] = jnp.where(msk, ex, 0.0)
            idxb[pl.ds(j * 16, 16)] = jnp.where(msk, dv - lo, 0)
        cp.wait()

        def _scale(i, cc):
            es = plsc.load_gather(exb, [jnp.full((16,), 0, jnp.int32) + i])
            for k in range(GW // 16):
                rows[i, pl.ds(k * 16, 16)] = rows[i, pl.ds(k * 16, 16)] * es
            return cc

        lax.fori_loop(0, EB, _scale, 0)
        pltpu.sync_copy(rows, num_sh.at[idxb], add=True)
        return c

    lax.fori_loop(0, nblk, _blk, 0)
    plsc.subcore_barrier()

    nch = jnp.minimum(STRIPE // CHUNK, (HALF - sid * STRIPE + CHUNK - 1) // CHUNK)

    def _wb(k, c):
        off = sid * STRIPE + k * CHUNK
        pltpu.sync_copy(num_sh.at[pl.ds(off, CHUNK)], zbuf)
        pltpu.sync_copy(zbuf, num_h.at[pl.ds(lo + off, CHUNK)])
        return c

    lax.fori_loop(0, nch, _wb, 0)


_SC_PARAMS = pltpu.CompilerParams(needs_layout_passes=False,
                                  use_tc_tiling_on_sc=False)


def _gat(g_tab, src, dst, a_s, a_d):
    mesh = plsc.VectorSubcoreMesh(core_axis_name="c", subcore_axis_name="s")
    ex = pl.kernel(
        _edge_ex_body,
        out_type=jax.ShapeDtypeStruct((E,), jnp.float32),
        mesh=mesh,
        compiler_params=_SC_PARAMS,
        scratch_types=[
            pltpu.VMEM((N,), jnp.float32),
            pltpu.VMEM((N,), jnp.float32),
            pltpu.VMEM((EB,), jnp.int32),
            pltpu.VMEM((EB,), jnp.int32),
            pltpu.VMEM((EB,), jnp.float32),
        ],
    )(src, dst, a_s, a_d)

    num = pl.kernel(
        _scatter_body,
        out_type=jax.ShapeDtypeStruct((N, GW), jnp.float32),
        mesh=mesh,
        compiler_params=_SC_PARAMS,
        scratch_types=[
            pltpu.VMEM((EB,), jnp.int32),
            pltpu.VMEM((EB,), jnp.int32),
            pltpu.VMEM((EB,), jnp.int32),
            pltpu.VMEM((EB,), jnp.float32),
            pltpu.VMEM((EB, GW), jnp.float32),
            pltpu.VMEM((CHUNK, GW), jnp.float32),
            pltpu.VMEM_SHARED((16 * STRIPE, GW), jnp.float32),
            pltpu.SemaphoreType.DMA,
        ],
    )(g_tab, src, dst, ex)
    return num


_LOG_SQRT_2PI = 0.9189385332046727


def _gammaln(z):
    p = z
    for k in range(1, 8):
        p = p * (z + float(k))
    w = z + 8.0
    inv = 1.0 / w
    inv2 = inv * inv
    series = inv * (1.0 / 12.0 - inv2 * (1.0 / 360.0 - inv2 * (1.0 / 1260.0)))
    return (w - 0.5) * jnp.log(w) - w + _LOG_SQRT_2PI + series - jnp.log(p)


def _dec_body(num_ref, cnt_ref, nrm_ref, lib_ref, bf_ref,
              wd1_ref, bd1_ref, wd2_ref, bd2_ref,
              wpi_ref, wdi_ref, wme_ref, wre_ref,
              xlat_ref, sums_ref, acc_ref):
    i = pl.program_id(0)
    numb = num_ref[...]
    xl = numb[:, 0:DL] / (numb[:, DEN_COL:DEN_COL + 1] + 1e-16)
    xlat_ref[...] = xl

    cls = lax.broadcasted_iota(jnp.int32, (BLK, NB), 1).astype(jnp.float32)
    oh = jnp.where(cls == bf_ref[...], 1.0, 0.0)
    z = jnp.concatenate([xl, oh, jnp.zeros((BLK, 2), jnp.float32)], axis=1)
    hd = jnp.maximum(jnp.dot(z, wd1_ref[...], preferred_element_type=jnp.float32)
                     + bd1_ref[...], 0.0)
    hd = jnp.maximum(jnp.dot(hd, wd2_ref[...], preferred_element_type=jnp.float32)
                     + bd2_ref[...], 0.0)
    pi = jax.nn.sigmoid(jnp.dot(hd, wpi_ref[...], preferred_element_type=jnp.float32))
    disp = jnp.exp(jnp.clip(jnp.dot(hd, wdi_ref[...], preferred_element_type=jnp.float32),
                            -15.0, 15.0))
    mean = jnp.exp(jnp.clip(jnp.dot(hd, wme_ref[...], preferred_element_type=jnp.float32),
                            -15.0, 15.0))
    recons = jnp.dot(hd, wre_ref[...], preferred_element_type=jnp.float32)

    mu = mean * lib_ref[...]
    eps = 1e-10
    x = cnt_ref[...]
    t1 = _gammaln(disp + eps) + _gammaln(x + 1.0) - _gammaln(x + disp + eps)
    t2 = ((disp + x) * jnp.log1p(mu / (disp + eps))
          + x * (jnp.log(disp + eps) - jnp.log(mu + eps)))
    nb_case = t1 + t2 - jnp.log(1.0 - pi + eps)
    zero_nb = jnp.exp(disp * jnp.log(disp / (disp + mu + eps)))
    zero_case = -jnp.log(pi + (1.0 - pi) * zero_nb + eps)
    res = jnp.where(x < 1e-8, zero_case, nb_case)
    nll_p = jnp.sum(res + 0.5 * pi * pi)
    mse_p = jnp.sum(jnp.square(recons - nrm_ref[...]))

    @pl.when(i == 0)
    def _():
        acc_ref[0, 0] = 0.0
        acc_ref[0, 1] = 0.0

    acc_ref[0, 0] += nll_p
    acc_ref[0, 1] += mse_p

    @pl.when(i == pl.num_programs(0) - 1)
    def _():
        sums_ref[...] = (jnp.stack([acc_ref[0, 0], acc_ref[0, 1]])
                         .reshape(1, 2) / float(N * D))


def _decoder(num, rna_counts, rna_norm, rna_libsize, batch_f,
             wd1_pad, bd1, wd2, bd2, wpi, wdisp, wmean, wrec):
    nb = N // BLK
    full = lambda i: (0, 0)
    return pl.pallas_call(
        _dec_body,
        grid=(nb,),
        in_specs=[
            pl.BlockSpec((BLK, GW), lambda i: (i, 0)),
            pl.BlockSpec((BLK, D), lambda i: (i, 0)),
            pl.BlockSpec((BLK, D), lambda i: (i, 0)),
            pl.BlockSpec((BLK, 1), lambda i: (i, 0)),
            pl.BlockSpec((BLK, 1), lambda i: (i, 0)),
            pl.BlockSpec((DL + NB + 2, H2), full),
            pl.BlockSpec((1, H2), full),
            pl.BlockSpec((H2, H1), full),
            pl.BlockSpec((1, H1), full),
            pl.BlockSpec((H1, D), full),
            pl.BlockSpec((H1, D), full),
            pl.BlockSpec((H1, D), full),
            pl.BlockSpec((H1, D), full),
        ],
        out_specs=[
            pl.BlockSpec((BLK, DL), lambda i: (i, 0)),
            pl.BlockSpec((1, 2), full),
        ],
        out_shape=[
            jax.ShapeDtypeStruct((N, DL), jnp.float32),
            jax.ShapeDtypeStruct((1, 2), jnp.float32),
        ],
        scratch_shapes=[pltpu.SMEM((1, 2), jnp.float32)],
    )(num, rna_counts, rna_norm, rna_libsize, batch_f,
      wd1_pad, bd1, wd2, bd2, wpi, wdisp, wmean, wrec)


@jax.jit
def kernel(rna_norm, rna_counts, rna_libsize, cas_norm, cas_counts, cas_libsize,
           adt_norm, edge_index, batch_indices,
           W1, b1, W2, b2, Wg, a_src, a_dst, batch_emb,
           Wd1, bd1, Wd2, bd2, Wpi, Wdisp, Wmean, Wrec):
    wg_pad = jnp.zeros((H2, GW), jnp.float32).at[:, :DL].set(Wg)
    asrc_pad = jnp.zeros((GW, 1), jnp.float32).at[:DL, 0].set(a_src)
    adst_pad = jnp.zeros((GW, 1), jnp.float32).at[:DL, 0].set(a_dst)
    g_tab, a_s, a_d = _encoder(rna_norm, W1, b1.reshape(1, H1), W2,
                               b2.reshape(1, H2), wg_pad, asrc_pad, adst_pad)

    src = edge_index[0]
    dst = edge_index[1]
    num = _gat(g_tab, src, dst, a_s.reshape(N), a_d.reshape(N))

    wd1_pad = jnp.zeros((DL + NB + 2, H2), jnp.float32).at[:DL + NB, :].set(Wd1)
    batch_f = batch_indices.astype(jnp.float32).reshape(N, 1)
    x_lat, sums = _decoder(num, rna_counts, rna_norm, rna_libsize, batch_f,
                           wd1_pad, bd1.reshape(1, H2), Wd2, bd2.reshape(1, H1),
                           Wpi, Wdisp, Wmean, Wrec)
    zero = jnp.zeros((), jnp.float32)
    return (x_lat, sums[0, 0], sums[0, 1], zero, zero, zero)


# speedup vs baseline: 33.1674x; 2.0895x over previous
import functools

import jax
import jax.numpy as jnp
from jax import lax
from jax.experimental import pallas as pl
from jax.experimental.pallas import tpu as pltpu
from jax.experimental.pallas import tpu_sc as plsc

N = 50000
E = 800000
D = 128
DL = 50
NB = 4
H1 = 256
H2 = 128
GW = 64
DEN_COL = 50
HALF = N // 2
BLK = 1000
EB1 = 256
NBLK1 = E // EB1
RCAP = 25216
NREC = 64 * RCAP
STRIPE = 1600
CHUNK = 100


def _enc_body(x_ref, w1_ref, b1_ref, w2_ref, b2_ref, wg_ref, asr_ref, adr_ref,
              g_ref, as_ref, ad_ref):
    x = x_ref[...]
    h = jnp.maximum(jnp.dot(x, w1_ref[...], preferred_element_type=jnp.float32)
                    + b1_ref[...], 0.0)
    h = jnp.maximum(jnp.dot(h, w2_ref[...], preferred_element_type=jnp.float32)
                    + b2_ref[...], 0.0)
    g = jnp.dot(h, wg_ref[...], preferred_element_type=jnp.float32)
    col = lax.broadcasted_iota(jnp.int32, (BLK, GW), 1)
    g = jnp.where(col == DEN_COL, 1.0, g)
    g_ref[...] = g
    as_ref[...] = jnp.dot(g, asr_ref[...], preferred_element_type=jnp.float32)
    ad_ref[...] = jnp.dot(g, adr_ref[...], preferred_element_type=jnp.float32)


def _encoder(rna_norm, w1, b1, w2, b2, wg_pad, asrc_pad, adst_pad):
    nb = N // BLK
    full = lambda i: (0, 0)
    return pl.pallas_call(
        _enc_body,
        grid=(nb,),
        in_specs=[
            pl.BlockSpec((BLK, D), lambda i: (i, 0)),
            pl.BlockSpec((D, H1), full),
            pl.BlockSpec((1, H1), full),
            pl.BlockSpec((H1, H2), full),
            pl.BlockSpec((1, H2), full),
            pl.BlockSpec((H2, GW), full),
            pl.BlockSpec((GW, 1), full),
            pl.BlockSpec((GW, 1), full),
        ],
        out_specs=[
            pl.BlockSpec((BLK, GW), lambda i: (i, 0)),
            pl.BlockSpec((BLK, 1), lambda i: (i, 0)),
            pl.BlockSpec((BLK, 1), lambda i: (i, 0)),
        ],
        out_shape=[
            jax.ShapeDtypeStruct((N, GW), jnp.float32),
            jax.ShapeDtypeStruct((N, 1), jnp.float32),
            jax.ShapeDtypeStruct((N, 1), jnp.float32),
        ],
    )(rna_norm, w1, b1, w2, b2, wg_pad, asrc_pad, adst_pad)


def _edge_part_body(src_h, dst_h, as_h, ad_h, srcP, dstP, exP, cntH,
                    asb, adb, srcb, dstb, stg_s, stg_d, stg_e, cvm,
                    sem_in, fl0, fl1):
    cid = lax.axis_index("c")
    sid = lax.axis_index("s")
    wid = sid * 2 + cid

    pltpu.sync_copy(as_h, asb)
    pltpu.sync_copy(ad_h, adb)

    neg = jnp.full((16,), -3.0e38, jnp.float32)

    def _mx(i, carry):
        ms, md = carry
        ms = jnp.maximum(ms, asb[pl.ds(i * 16, 16)])
        md = jnp.maximum(md, adb[pl.ds(i * 16, 16)])
        return ms, md

    ms_v, md_v = lax.fori_loop(0, N // 16, _mx, (neg, neg))
    t = jnp.max(ms_v) + jnp.max(md_v)
    m_shift = jnp.where(t >= 0.0, t, 0.2 * t)

    nfull = NBLK1 // 32
    nblk = nfull + jnp.where(wid < NBLK1 - nfull * 32, 1, 0)
    sems = (fl0, fl1)

    def _fire_in(g, p):
        base = (g * 32 + wid) * EB1
        pltpu.async_copy(src_h.at[pl.ds(base, EB1)], srcb.at[p], sem_in)
        pltpu.async_copy(dst_h.at[pl.ds(base, EB1)], dstb.at[p], sem_in)

    _fire_in(0, 0)

    def _flush(h, q, f):
        off = (wid * 2 + h) * RCAP + f * 128
        pltpu.async_copy(stg_s.at[h, q, pl.ds(0, 128)],
                         srcP.at[pl.ds(off, 128)], sems[h])
        pltpu.async_copy(stg_d.at[h, q, pl.ds(0, 128)],
                         dstP.at[pl.ds(off, 128)], sems[h])
        pltpu.async_copy(stg_e.at[h, q, pl.ds(0, 128)],
                         exP.at[pl.ds(off, 128)], sems[h])

    def _drain(h):
        pltpu.make_async_copy(stg_s.at[0, 0, pl.ds(0, 128)],
                              srcP.at[pl.ds(0, 128)], sems[h]).wait()
        pltpu.make_async_copy(stg_d.at[0, 0, pl.ds(0, 128)],
                              dstP.at[pl.ds(0, 128)], sems[h]).wait()
        pltpu.make_async_copy(stg_e.at[0, 0, pl.ds(0, 128)],
                              exP.at[pl.ds(0, 128)], sems[h]).wait()

    def _bucket(h, msk, sv, dloc, ex, cnt, f):
        c = jnp.sum(jnp.where(msk, 1, 0))
        q = jnp.bitwise_and(f, 3)
        plsc.store_compressed(stg_s.at[h, q, pl.ds(cnt, 16)], sv, mask=msk)
        plsc.store_compressed(stg_d.at[h, q, pl.ds(cnt, 16)], dloc, mask=msk)
        plsc.store_compressed(stg_e.at[h, q, pl.ds(cnt, 16)], ex, mask=msk)
        cnt2 = cnt + c
        do = cnt2 >= 128

        @pl.when(do)
        def _():
            @pl.when(f >= 3)
            def _():
                _drain(h)

            _flush(h, q, f)
            q2 = jnp.bitwise_and(f + 1, 3)
            stg_s[h, q2, pl.ds(0, 16)] = stg_s[h, q, pl.ds(128, 16)]
            stg_d[h, q2, pl.ds(0, 16)] = stg_d[h, q, pl.ds(128, 16)]
            stg_e[h, q2, pl.ds(0, 16)] = stg_e[h, q, pl.ds(128, 16)]

        return jnp.where(do, cnt2 - 128, cnt2), jnp.where(do, f + 1, f)

    def _blk(g, carry):
        cnt0, cnt1, f0, f1 = carry
        p = jnp.bitwise_and(g, 1)
        pltpu.make_async_copy(src_h.at[pl.ds(0, EB1)], srcb.at[p], sem_in).wait()
        pltpu.make_async_copy(dst_h.at[pl.ds(0, EB1)], dstb.at[p], sem_in).wait()

        @pl.when(g + 1 < nblk)
        def _():
            _fire_in(g + 1, 1 - p)

        for j in range(EB1 // 16):
            sv = srcb[p, pl.ds(j * 16, 16)]
            dv = dstb[p, pl.ds(j * 16, 16)]
            e = plsc.load_gather(asb, [sv]) + plsc.load_gather(adb, [dv])
            e = jnp.where(e >= 0.0, e, 0.2 * e)
            ex = jnp.exp(e - m_shift)
            m0 = dv < HALF
            cnt0, f0 = _bucket(0, m0, sv, dv, ex, cnt0, f0)
            cnt1, f1 = _bucket(1, ~m0, sv, dv - HALF, ex, cnt1, f1)
        return cnt0, cnt1, f0, f1

    z = jnp.int32(0)
    cnt0, cnt1, f0, f1 = lax.fori_loop(0, nblk, _blk, (z, z, z, z))

    zi = jnp.zeros((16,), jnp.int32)
    zf = jnp.zeros((16,), jnp.float32)

    def _tail(h, cnt, f):
        q = jnp.bitwise_and(f, 3)
        for i in range(8):
            off = jnp.minimum(cnt + i * 16, 128)
            stg_s[h, q, pl.ds(off, 16)] = zi
            stg_d[h, q, pl.ds(off, 16)] = zi
            stg_e[h, q, pl.ds(off, 16)] = zf

        @pl.when(f >= 3)
        def _():
            _drain(h)

        _flush(h, q, f)
        return f + 1

    tf0 = _tail(0, cnt0, f0)
    tf1 = _tail(1, cnt1, f1)

    def _dr0(i, c):
        _drain(0)
        return c

    def _dr1(i, c):
        _drain(1)
        return c

    lax.fori_loop(0, jnp.minimum(tf0, 3), _dr0, 0)
    lax.fori_loop(0, jnp.minimum(tf1, 3), _dr1, 0)

    i16 = lax.iota(jnp.int32, 16)
    cvm[pl.ds(0, 16)] = jnp.where(i16 == 0, tf0, jnp.where(i16 == 1, tf1, 0))
    pltpu.sync_copy(cvm, cntH.at[wid])


def _scatter_body(g_h, srcP, dstP, exP, cntH, num_h,
                  srcb, idxb, exb, rows, zbuf, cvm, num_sh,
                  sem_src, sem_in, sem, sem_sc):
    cid = lax.axis_index("c")
    sid = lax.axis_index("s")
    lo = cid * HALF

    zeros16 = jnp.zeros((16,), jnp.float32)

    def _zrow(r, c):
        for k in range(GW // 16):
            zbuf[r, pl.ds(k * 16, 16)] = zeros16
        return c

    lax.fori_loop(0, CHUNK, _zrow, 0)

    def _zcp(k, c):
        pltpu.sync_copy(zbuf, num_sh.at[pl.ds(sid * STRIPE + k * CHUNK, CHUNK)])
        return c

    lax.fori_loop(0, STRIPE // CHUNK, _zcp, 0)
    plsc.subcore_barrier()

    wa = 2 * sid
    wb = 2 * sid + 1
    pltpu.sync_copy(cntH.at[wa], cvm.at[0])
    pltpu.sync_copy(cntH.at[wb], cvm.at[1])
    i16 = lax.iota(jnp.int32, 16)
    nA = jnp.sum(jnp.where(i16 == cid, cvm[0, pl.ds(0, 16)], 0))
    nB = jnp.sum(jnp.where(i16 == cid, cvm[1, pl.ds(0, 16)], 0))
    ntot = nA + nB
    ra = (wa * 2 + cid) * RCAP
    rb = (wb * 2 + cid) * RCAP

    def _base(g):
        return jnp.where(g < nA, ra + g * 128, rb + (g - nA) * 128)

    def _fire_in(g):
        b = _base(g)
        pltpu.async_copy(srcP.at[pl.ds(b, 128)], srcb.at[lax.rem(g, 2)],
                         sem_src)
        pltpu.async_copy(dstP.at[pl.ds(b, 128)], idxb.at[lax.rem(g, 3)],
                         sem_in)
        pltpu.async_copy(exP.at[pl.ds(b, 128)], exb.at[lax.rem(g, 3)],
                         sem_in)

    def _fire_gather(g):
        pltpu.async_copy(g_h.at[srcb.at[lax.rem(g, 2)]],
                         rows.at[lax.rem(g, 2)], sem)

    def _wait_src():
        pltpu.make_async_copy(srcP.at[pl.ds(0, 128)], srcb.at[0],
                              sem_src).wait()

    def _wait_inputs():
        pltpu.make_async_copy(dstP.at[pl.ds(0, 128)], idxb.at[0],
                              sem_in).wait()
        pltpu.make_async_copy(exP.at[pl.ds(0, 128)], exb.at[0],
                              sem_in).wait()

    def _wait_gather():
        pltpu.make_async_copy(g_h.at[srcb.at[0]], rows.at[0], sem).wait()

    def _drain_scatter():
        pltpu.make_async_copy(rows.at[0], num_sh.at[idxb.at[0]],
                              sem_sc).wait()

    _fire_in(0)
    _fire_in(1)
    _wait_src()
    _fire_gather(0)

    def _blk(g, c):
        p2 = lax.rem(g, 2)
        p3 = lax.rem(g, 3)
        _wait_inputs()
        _wait_gather()

        @pl.when(g >= 1)
        def _():
            _drain_scatter()

        @pl.when(g + 1 < ntot)
        def _():
            _wait_src()
            _fire_gather(g + 1)

        def _scale(i, cc):
            for u in range(8):
                r = i * 8 + u
                es = plsc.load_gather(exb.at[p3],
                                      [jnp.full((16,), 0, jnp.int32) + r])
                for k in range(GW // 16):
                    rows[p2, r, pl.ds(k * 16, 16)] = (
                        rows[p2, r, pl.ds(k * 16, 16)] * es)
            return cc

        lax.fori_loop(0, 128 // 8, _scale, 0)
        pltpu.async_copy(rows.at[p2], num_sh.at[idxb.at[p3]], sem_sc,
                         add=True)

        @pl.when(g + 2 < ntot)
        def _():
            _fire_in(g + 2)

        return c

    lax.fori_loop(0, ntot, _blk, 0)
    _drain_scatter()
    plsc.subcore_barrier()

    nch = jnp.minimum(STRIPE // CHUNK, (HALF - sid * STRIPE + CHUNK - 1) // CHUNK)

    def _wb(k, c):
        off = sid * STRIPE + k * CHUNK
        pltpu.sync_copy(num_sh.at[pl.ds(off, CHUNK)], zbuf)
        pltpu.sync_copy(zbuf, num_h.at[pl.ds(lo + off, CHUNK)])
        return c

    lax.fori_loop(0, nch, _wb, 0)


_SC_PARAMS = pltpu.CompilerParams(needs_layout_passes=False,
                                  use_tc_tiling_on_sc=False)


def _gat(g_tab, src, dst, a_s, a_d):
    mesh = plsc.VectorSubcoreMesh(core_axis_name="c", subcore_axis_name="s")
    srcP, dstP, exP, cntH = pl.kernel(
        _edge_part_body,
        out_type=[
            jax.ShapeDtypeStruct((NREC,), jnp.int32),
            jax.ShapeDtypeStruct((NREC,), jnp.int32),
            jax.ShapeDtypeStruct((NREC,), jnp.float32),
            jax.ShapeDtypeStruct((32, 16), jnp.int32),
        ],
        mesh=mesh,
        compiler_params=_SC_PARAMS,
        scratch_types=[
            pltpu.VMEM((N,), jnp.float32),
            pltpu.VMEM((N,), jnp.float32),
            pltpu.VMEM((2, EB1), jnp.int32),
            pltpu.VMEM((2, EB1), jnp.int32),
            pltpu.VMEM((2, 4, 160), jnp.int32),
            pltpu.VMEM((2, 4, 160), jnp.int32),
            pltpu.VMEM((2, 4, 160), jnp.float32),
            pltpu.VMEM((16,), jnp.int32),
            pltpu.SemaphoreType.DMA,
            pltpu.SemaphoreType.DMA,
            pltpu.SemaphoreType.DMA,
        ],
    )(src, dst, a_s, a_d)

    num = pl.kernel(
        _scatter_body,
        out_type=jax.ShapeDtypeStruct((N, GW), jnp.float32),
        mesh=mesh,
        compiler_params=_SC_PARAMS,
        scratch_types=[
            pltpu.VMEM((2, 128), jnp.int32),
            pltpu.VMEM((3, 128), jnp.int32),
            pltpu.VMEM((3, 128), jnp.float32),
            pltpu.VMEM((2, 128, GW), jnp.float32),
            pltpu.VMEM((CHUNK, GW), jnp.float32),
            pltpu.VMEM((2, 16), jnp.int32),
            pltpu.VMEM_SHARED((16 * STRIPE, GW), jnp.float32),
            pltpu.SemaphoreType.DMA,
            pltpu.SemaphoreType.DMA,
            pltpu.SemaphoreType.DMA,
            pltpu.SemaphoreType.DMA,
        ],
    )(g_tab, srcP, dstP, exP, cntH)
    return num


_LOG_SQRT_2PI = 0.9189385332046727


def _stirl(w):
    inv = 1.0 / w
    inv2 = inv * inv
    series = inv * (1.0 / 12.0 - inv2 * (1.0 / 360.0 - inv2 * (1.0 / 1260.0)))
    return (w - 0.5) * jnp.log(w) - w + _LOG_SQRT_2PI + series


def _p4(z):
    return z * (z + 1.0) * (z + 2.0) * (z + 3.0)


def _dec_body(num_ref, cnt_ref, nrm_ref, lib_ref, bf_ref,
              wd1_ref, bd1_ref, wd2_ref, bd2_ref,
              wpi_ref, wdi_ref, wme_ref, wre_ref,
              xlat_ref, sums_ref, acc_ref):
    i = pl.program_id(0)
    numb = num_ref[...]
    xl = numb[:, 0:DL] / (numb[:, DEN_COL:DEN_COL + 1] + 1e-16)
    xlat_ref[...] = xl

    cls = lax.broadcasted_iota(jnp.int32, (BLK, NB), 1).astype(jnp.float32)
    oh = jnp.where(cls == bf_ref[...], 1.0, 0.0)
    z = jnp.concatenate([xl, oh, jnp.zeros((BLK, 2), jnp.float32)], axis=1)
    hd = jnp.maximum(jnp.dot(z, wd1_ref[...], preferred_element_type=jnp.float32)
                     + bd1_ref[...], 0.0)
    hd = jnp.maximum(jnp.dot(hd, wd2_ref[...], preferred_element_type=jnp.float32)
                     + bd2_ref[...], 0.0)
    pi = jax.nn.sigmoid(jnp.dot(hd, wpi_ref[...], preferred_element_type=jnp.float32))
    disp = jnp.exp(jnp.clip(jnp.dot(hd, wdi_ref[...], preferred_element_type=jnp.float32),
                            -15.0, 15.0))
    mean = jnp.exp(jnp.clip(jnp.dot(hd, wme_ref[...], preferred_element_type=jnp.float32),
                            -15.0, 15.0))
    recons = jnp.dot(hd, wre_ref[...], preferred_element_type=jnp.float32)

    mu = mean * lib_ref[...]
    eps = 1e-10
    x = cnt_ref[...]
    dd = disp + eps
    xd = x + dd
    t1 = (_stirl(dd + 4.0) + _stirl(x + 5.0) - _stirl(xd + 4.0)
          - jnp.log(_p4(dd) * _p4(x + 1.0) / _p4(xd)))
    t2 = ((disp + x) * jnp.log1p(mu / dd)
          + x * jnp.log(dd / (mu + eps)))
    nb_case = t1 + t2 - jnp.log(1.0 - pi + eps)
    zero_nb = jnp.exp(disp * jnp.log(disp / (disp + mu + eps)))
    zero_case = -jnp.log(pi + (1.0 - pi) * zero_nb + eps)
    res = jnp.where(x < 1e-8, zero_case, nb_case)
    nll_p = jnp.sum(res + 0.5 * pi * pi)
    mse_p = jnp.sum(jnp.square(recons - nrm_ref[...]))

    @pl.when(i == 0)
    def _():
        acc_ref[0, 0] = 0.0
        acc_ref[0, 1] = 0.0

    acc_ref[0, 0] += nll_p
    acc_ref[0, 1] += mse_p

    @pl.when(i == pl.num_programs(0) - 1)
    def _():
        sums_ref[...] = (jnp.stack([acc_ref[0, 0], acc_ref[0, 1]])
                         .reshape(1, 2) / float(N * D))


def _decoder(num, rna_counts, rna_norm, rna_libsize, batch_f,
             wd1_pad, bd1, wd2, bd2, wpi, wdisp, wmean, wrec):
    nb = N // BLK
    full = lambda i: (0, 0)
    return pl.pallas_call(
        _dec_body,
        grid=(nb,),
        in_specs=[
            pl.BlockSpec((BLK, GW), lambda i: (i, 0)),
            pl.BlockSpec((BLK, D), lambda i: (i, 0)),
            pl.BlockSpec((BLK, D), lambda i: (i, 0)),
            pl.BlockSpec((BLK, 1), lambda i: (i, 0)),
            pl.BlockSpec((BLK, 1), lambda i: (i, 0)),
            pl.BlockSpec((DL + NB + 2, H2), full),
            pl.BlockSpec((1, H2), full),
            pl.BlockSpec((H2, H1), full),
            pl.BlockSpec((1, H1), full),
            pl.BlockSpec((H1, D), full),
            pl.BlockSpec((H1, D), full),
            pl.BlockSpec((H1, D), full),
            pl.BlockSpec((H1, D), full),
        ],
        out_specs=[
            pl.BlockSpec((BLK, DL), lambda i: (i, 0)),
            pl.BlockSpec((1, 2), full),
        ],
        out_shape=[
            jax.ShapeDtypeStruct((N, DL), jnp.float32),
            jax.ShapeDtypeStruct((1, 2), jnp.float32),
        ],
        scratch_shapes=[pltpu.SMEM((1, 2), jnp.float32)],
    )(num, rna_counts, rna_norm, rna_libsize, batch_f,
      wd1_pad, bd1, wd2, bd2, wpi, wdisp, wmean, wrec)


@jax.jit
def kernel(rna_norm, rna_counts, rna_libsize, cas_norm, cas_counts, cas_libsize,
           adt_norm, edge_index, batch_indices,
           W1, b1, W2, b2, Wg, a_src, a_dst, batch_emb,
           Wd1, bd1, Wd2, bd2, Wpi, Wdisp, Wmean, Wrec):
    wg_pad = jnp.zeros((H2, GW), jnp.float32).at[:, :DL].set(Wg)
    asrc_pad = jnp.zeros((GW, 1), jnp.float32).at[:DL, 0].set(a_src)
    adst_pad = jnp.zeros((GW, 1), jnp.float32).at[:DL, 0].set(a_dst)
    g_tab, a_s, a_d = _encoder(rna_norm, W1, b1.reshape(1, H1), W2,
                               b2.reshape(1, H2), wg_pad, asrc_pad, adst_pad)

    src = edge_index[0]
    dst = edge_index[1]
    num = _gat(g_tab, src, dst, a_s.reshape(N), a_d.reshape(N))

    wd1_pad = jnp.zeros((DL + NB + 2, H2), jnp.float32).at[:DL + NB, :].set(Wd1)
    batch_f = batch_indices.astype(jnp.float32).reshape(N, 1)
    x_lat, sums = _decoder(num, rna_counts, rna_norm, rna_libsize, batch_f,
                           wd1_pad, bd1.reshape(1, H2), Wd2, bd2.reshape(1, H1),
                           Wpi, Wdisp, Wmean, Wrec)
    zero = jnp.zeros((), jnp.float32)
    return (x_lat, sums[0, 0], sums[0, 1], zero, zero, zero)
